# Initial kernel scaffold; baseline (speedup 1.0000x reference)
import jax
import jax.numpy as jnp
from jax.experimental import pallas as pl


def kernel(x, pos_table):
    raise NotImplementedError("write your pallas kernel here")


# speedup vs baseline: 3.4048x; 3.4048x over previous
import functools

import jax
import jax.numpy as jnp
from jax import lax
from jax.experimental import pallas as pl
from jax.experimental.pallas import tpu as pltpu
from jax.experimental.pallas import tpu_sc as plsc

NC = 2
NS = 16
NW = NC * NS

B = 4
S = 8192
D = 1024
ROWS_PER_W = S // NW
CHUNK = 32
N_CHUNKS = ROWS_PER_W // CHUNK


def _make_sc_copy():
    mesh = plsc.VectorSubcoreMesh(core_axis_name="c", subcore_axis_name="s")

    @functools.partial(
        pl.kernel,
        out_type=jax.ShapeDtypeStruct((B, S, D), jnp.float32),
        mesh=mesh,
        scratch_types=[
            pltpu.VMEM((CHUNK, D), jnp.float32),
        ],
    )
    def body(table_hbm, out_hbm, buf):
        wid = lax.axis_index("s") * NC + lax.axis_index("c")
        base = wid * ROWS_PER_W
        for i in range(N_CHUNKS):
            r0 = base + i * CHUNK
            pltpu.sync_copy(table_hbm.at[pl.ds(r0, CHUNK)], buf)
            for b in range(B):
                pltpu.sync_copy(buf, out_hbm.at[b, pl.ds(r0, CHUNK)])

    return body


_sc_copy = _make_sc_copy()


def kernel(x, pos_table):
    del x
    return _sc_copy(pos_table)


# speedup vs baseline: 3.4097x; 1.0014x over previous
import functools

import jax
import jax.numpy as jnp
from jax import lax
from jax.experimental import pallas as pl
from jax.experimental.pallas import tpu as pltpu
from jax.experimental.pallas import tpu_sc as plsc

NC = 2
NS = 16
NW = NC * NS

B = 4
S = 8192
D = 1024
ROWS_PER_W = S // NW
CHUNK = 32
N_CHUNKS = ROWS_PER_W // CHUNK


def _make_sc_copy():
    mesh = plsc.VectorSubcoreMesh(core_axis_name="c", subcore_axis_name="s")

    @functools.partial(
        pl.kernel,
        out_type=jax.ShapeDtypeStruct((B, S, D), jnp.float32),
        mesh=mesh,
        scratch_types=[
            pltpu.VMEM((CHUNK, D), jnp.float32),
            pltpu.VMEM((CHUNK, D), jnp.float32),
            pltpu.SemaphoreType.DMA,
            pltpu.SemaphoreType.DMA,
            pltpu.SemaphoreType.DMA,
            pltpu.SemaphoreType.DMA,
        ],
    )
    def body(table_hbm, out_hbm, buf0, buf1, isem0, isem1, osem0, osem1):
        wid = lax.axis_index("s") * NC + lax.axis_index("c")
        base = wid * ROWS_PER_W
        bufs = (buf0, buf1)
        isems = (isem0, isem1)
        osems = (osem0, osem1)

        in_handles = [None] * N_CHUNKS
        out_handles = [[] for _ in range(N_CHUNKS)]

        def start_in(i):
            r0 = base + i * CHUNK
            in_handles[i] = pltpu.async_copy(
                table_hbm.at[pl.ds(r0, CHUNK)], bufs[i % 2], isems[i % 2])

        start_in(0)
        for i in range(N_CHUNKS):
            cur = i % 2
            in_handles[i].wait()
            r0 = base + i * CHUNK
            for b in range(B):
                out_handles[i].append(pltpu.async_copy(
                    bufs[cur], out_hbm.at[b, pl.ds(r0, CHUNK)], osems[cur]))
            if i + 1 < N_CHUNKS:
                if i >= 1:
                    for h in out_handles[i - 1]:
                        h.wait()
                start_in(i + 1)
        for i in (N_CHUNKS - 2, N_CHUNKS - 1):
            for h in out_handles[i]:
                h.wait()

    return body


_sc_copy = _make_sc_copy()


def kernel(x, pos_table):
    del x
    return _sc_copy(pos_table)
